# Initial kernel scaffold; baseline (speedup 1.0000x reference)
import jax
import jax.numpy as jnp
from jax.experimental import pallas as pl


def kernel(x, edge_attr, edge_index, line_graph_edge_index, edge_index_batch, W_mlp, b_mlp, W_u, W_v, W_edge, W_att_root, W_att_rel, b_att, a, W_gout, b_gout, a_bias, W_lb, b_lb):
    raise NotImplementedError("write your pallas kernel here")


# speedup vs baseline: 2.1281x; 2.1281x over previous
import functools

import jax
import jax.numpy as jnp
from jax import lax
from jax.experimental import pallas as pl
from jax.experimental.pallas import tpu as pltpu
from jax.experimental.pallas import tpu_sc as plsc

N = 10000
E = 160000
L = 320000
B = 256
D = 128
NITER = 4

NC = 2
NS = 16
NW = NC * NS

BE = 1280
NBLK_E = E // BE
BN = 1000
NBLK_N = N // BN

K = 128
CH = 10000
NCHUNK = E // CH
NROUND = NCHUNK // NC
LT = L // NS
LTP = LT + 160
PIECE = 2000

@functools.cache
def _mesh():
    return plsc.VectorSubcoreMesh(core_axis_name="c", subcore_axis_name="s",
                                  num_cores=NC, num_subcores=NS)

_NEG = -1e30


def _prep_node_body(x_ref, wm_ref, bm_ref, wu_ref, wv_ref, h_ref, eu_ref, ev_ref):
    h = jnp.dot(x_ref[...], wm_ref[...], preferred_element_type=jnp.float32)
    h = h + bm_ref[...]
    h_ref[...] = h
    third = jnp.float32(1.0 / 3.0)
    eu_ref[...] = jnp.dot(h, wu_ref[...], preferred_element_type=jnp.float32) * third
    ev_ref[...] = jnp.dot(h, wv_ref[...], preferred_element_type=jnp.float32) * third


def _prep_node(x, W_mlp, b_mlp2, W_u, W_v):
    return pl.pallas_call(
        _prep_node_body,
        grid=(NBLK_N,),
        in_specs=[
            pl.BlockSpec((BN, D), lambda g: (g, 0)),
            pl.BlockSpec((D, D), lambda g: (0, 0)),
            pl.BlockSpec((1, D), lambda g: (0, 0)),
            pl.BlockSpec((D, D), lambda g: (0, 0)),
            pl.BlockSpec((D, D), lambda g: (0, 0)),
        ],
        out_specs=[
            pl.BlockSpec((BN, D), lambda g: (g, 0)),
            pl.BlockSpec((BN, D), lambda g: (g, 0)),
            pl.BlockSpec((BN, D), lambda g: (g, 0)),
        ],
        out_shape=[jax.ShapeDtypeStruct((N, D), jnp.float32)] * 3,
    )(x, W_mlp, b_mlp2, W_u, W_v)


def _prep_edge_body(ea_ref, we_ref, o_ref):
    o_ref[...] = jnp.dot(ea_ref[...], we_ref[...],
                         preferred_element_type=jnp.float32) * jnp.float32(1.0 / 3.0)


def _prep_edge(edge_attr, W_edge):
    ed = edge_attr.shape[1]
    return pl.pallas_call(
        _prep_edge_body,
        grid=(NBLK_E,),
        in_specs=[
            pl.BlockSpec((BE, ed), lambda g: (g, 0)),
            pl.BlockSpec((ed, D), lambda g: (0, 0)),
        ],
        out_specs=pl.BlockSpec((BE, D), lambda g: (g, 0)),
        out_shape=jax.ShapeDtypeStruct((E, D), jnp.float32),
    )(edge_attr, W_edge)


@functools.cache
def _gather2_fn():
    return functools.partial(
        pl.kernel,
        out_type=(jax.ShapeDtypeStruct((E, D), jnp.float32),
                  jax.ShapeDtypeStruct((E, D), jnp.float32)),
        mesh=_mesh(),
        compiler_params=pltpu.CompilerParams(needs_layout_passes=False),
        scratch_types=[
            pltpu.VMEM((K,), jnp.int32),
            pltpu.VMEM((K,), jnp.int32),
            pltpu.VMEM((K, D), jnp.float32),
            pltpu.VMEM((K, D), jnp.float32),
            pltpu.SemaphoreType.DMA,
            pltpu.SemaphoreType.DMA,
        ],
    )(_gather2_body)


def _gather2_body(eu_hbm, ev_hbm, src_hbm, dst_hbm, geu_hbm, gev_hbm,
                  src_v, dst_v, bufa, bufb, sema, semb):
    wid = lax.axis_index("s") * NC + lax.axis_index("c")
    nbtot = E // K
    nfull = nbtot // NW
    nb = jnp.where(wid < (nbtot - nfull * NW), nfull + 1, nfull)

    def body(i, _):
        base = (i * NW + wid) * K
        pltpu.sync_copy(src_hbm.at[pl.ds(base, K)], src_v)
        pltpu.sync_copy(dst_hbm.at[pl.ds(base, K)], dst_v)
        ca = pltpu.async_copy(eu_hbm.at[src_v], bufa, sema)
        cb = pltpu.async_copy(ev_hbm.at[dst_v], bufb, semb)
        ca.wait()
        cb.wait()
        pltpu.sync_copy(bufa, geu_hbm.at[pl.ds(base, K)])
        pltpu.sync_copy(bufb, gev_hbm.at[pl.ds(base, K)])
        return ()

    lax.fori_loop(0, nb, body, (), unroll=False)


def _add3_body(a_ref, b_ref, c_ref, o_ref):
    o_ref[...] = a_ref[...] + b_ref[...] + c_ref[...]


def _add3(a, b, c):
    return pl.pallas_call(
        _add3_body,
        grid=(NBLK_E,),
        in_specs=[pl.BlockSpec((BE, D), lambda g: (g, 0))] * 3,
        out_specs=pl.BlockSpec((BE, D), lambda g: (g, 0)),
        out_shape=jax.ShapeDtypeStruct((E, D), jnp.float32),
    )(a, b, c)


@functools.cache
def _lg_pass_fn():
    return functools.partial(
        pl.kernel,
        out_type=jax.ShapeDtypeStruct((E, D), jnp.float32),
        mesh=_mesh(),
        compiler_params=pltpu.CompilerParams(needs_layout_passes=False),
        scratch_types=[
            pltpu.VMEM_SHARED((CH + 8, D), jnp.float32),
            pltpu.VMEM((PIECE,), jnp.int32),
            pltpu.VMEM((PIECE,), jnp.int32),
            pltpu.VMEM((LTP,), jnp.int32),
            pltpu.VMEM((K,), jnp.int32),
            pltpu.VMEM((K,), jnp.int32),
            pltpu.VMEM((K, D), jnp.float32),
            pltpu.SemaphoreType.DMA,
        ],
    )(_lg_pass_body)


_SRC_BITS = 18
_PADW = ((CH << _SRC_BITS) & 0xFFFFFFFF) - (1 << 32)


def _lg_pass_body(table_hbm, init_hbm, lgs_hbm, lgd_hbm, out_hbm,
                  shared, dpiece, spiece, pbuf, rel_stage, src_stage,
                  rows, semg):
    cid = lax.axis_index("c")
    sid = lax.axis_index("s")
    trows = 624
    ttail = CH - NS * trows

    def round_body(r, _):
        base = (r * NC + cid) * CH

        pltpu.sync_copy(init_hbm.at[pl.ds(base + sid * trows, trows)],
                        shared.at[pl.ds(sid * trows, trows)])

        @pl.when(sid == NS - 1)
        def _init_tail():
            pltpu.sync_copy(init_hbm.at[pl.ds(base + NS * trows, ttail)],
                            shared.at[pl.ds(NS * trows, ttail)])

        plsc.subcore_barrier()

        def piece_body(p, cnt):
            off = sid * LT + p * PIECE
            pltpu.sync_copy(lgd_hbm.at[pl.ds(off, PIECE)], dpiece)
            pltpu.sync_copy(lgs_hbm.at[pl.ds(off, PIECE)], spiece)

            def scan_body(j, cnt2):
                d16 = dpiece[pl.ds(j * 16, 16)]
                s16 = spiece[pl.ds(j * 16, 16)]
                rel = d16 - base
                msk = (rel >= 0) & (rel < CH)
                packed = jnp.where(msk, (rel << _SRC_BITS) | s16,
                                   jnp.int32(_PADW))
                keys = jnp.where(msk, jnp.int32(0), jnp.int32(1))
                _, sv = plsc.sort_key_val(keys, packed)
                pbuf[pl.ds(cnt2, 16)] = sv
                return cnt2 + plsc.all_reduce_population_count(msk)[0]

            return lax.fori_loop(0, PIECE // 16, scan_body, cnt,
                                 unroll=False)

        cnt = lax.fori_loop(0, LT // PIECE, piece_body, jnp.int32(0),
                            unroll=False)

        padv = jnp.full((16,), jnp.int32(_PADW), jnp.int32)
        for t in range(K // 16):
            pbuf[pl.ds(cnt + t * 16, 16)] = padv
        nb = (cnt + (K - 1)) // K

        def drain_body(bi, _):
            for t in range(K // 16):
                pk = pbuf[pl.ds(bi * K + t * 16, 16)]
                rel_stage[pl.ds(t * 16, 16)] = lax.shift_right_logical(
                    pk, _SRC_BITS)
                src_stage[pl.ds(t * 16, 16)] = pk & ((1 << _SRC_BITS) - 1)
            pltpu.async_copy(table_hbm.at[src_stage], rows, semg).wait()
            pltpu.sync_copy(rows, shared.at[rel_stage], add=True)
            return ()

        lax.fori_loop(0, nb, drain_body, (), unroll=False)
        plsc.subcore_barrier()

        pltpu.sync_copy(shared.at[pl.ds(sid * trows, trows)],
                        out_hbm.at[pl.ds(base + sid * trows, trows)])

        @pl.when(sid == NS - 1)
        def _out_tail():
            pltpu.sync_copy(shared.at[pl.ds(NS * trows, ttail)],
                            out_hbm.at[pl.ds(base + NS * trows, ttail)])

        return ()

    lax.fori_loop(0, NROUND, round_body, (), unroll=False)


def _xcstats_body(o1, o2, o3, o4, o5, e0r, wr, wl, ba, bat,
                  xct_ref, m_ref, den_ref, m_s, den_s):
    g = pl.program_id(0)

    @pl.when(g == 0)
    def _init():
        m_s[...] = jnp.full((NITER, B), _NEG, jnp.float32)
        den_s[...] = jnp.zeros((NITER, B), jnp.float32)

    outs = (o1[...], o2[...], o3[...], o4[...], o5[...])
    e0b = e0r[...]
    wrv = wr[...]
    wlv = wl[...]
    bav = ba[0, 0]
    ids = bat[0, 0]
    ohT = lax.broadcasted_iota(jnp.int32, (B, BE), 0) == ids[None, :]
    ohTf = ohT.astype(jnp.float32)

    xcs = []
    for n in range(NITER):
        xcn = (jnp.sum(outs[n] * wrv, axis=1)
               + jnp.sum((outs[n + 1] - e0b) * wlv, axis=1) + bav)
        xcs.append(xcn)
        contrib = jnp.max(jnp.where(ohT, xcn[None, :], _NEG), axis=1)
        mold = m_s[n]
        mnew = jnp.maximum(mold, contrib)
        mrow = jnp.dot(mnew, ohTf, preferred_element_type=jnp.float32)
        ex = jnp.exp(xcn - mrow)
        dc = jnp.dot(ohTf, ex, preferred_element_type=jnp.float32)
        den_s[n] = den_s[n] * jnp.exp(mold - mnew) + dc
        m_s[n] = mnew

    xct_ref[...] = jnp.stack(xcs, axis=0)

    @pl.when(g == NBLK_E - 1)
    def _fin():
        m_ref[...] = m_s[...]
        den_ref[...] = den_s[...]


def _xcstats(o1, o2, o3, o4, o5, e0, wr2, wl2, ba2, batch3):
    blk = pl.BlockSpec((BE, D), lambda g: (g, 0))
    return pl.pallas_call(
        _xcstats_body,
        grid=(NBLK_E,),
        in_specs=[blk, blk, blk, blk, blk, blk,
                  pl.BlockSpec((1, D), lambda g: (0, 0)),
                  pl.BlockSpec((1, D), lambda g: (0, 0)),
                  pl.BlockSpec((1, 1), lambda g: (0, 0)),
                  pl.BlockSpec((1, 1, BE), lambda g: (g, 0, 0))],
        out_specs=[pl.BlockSpec((NITER, BE), lambda g: (0, g)),
                   pl.BlockSpec((NITER, B), lambda g: (0, 0)),
                   pl.BlockSpec((NITER, B), lambda g: (0, 0))],
        out_shape=[jax.ShapeDtypeStruct((NITER, E), jnp.float32),
                   jax.ShapeDtypeStruct((NITER, B), jnp.float32),
                   jax.ShapeDtypeStruct((NITER, B), jnp.float32)],
        scratch_shapes=[pltpu.VMEM((NITER, B), jnp.float32),
                        pltpu.VMEM((NITER, B), jnp.float32)],
    )(o1, o2, o3, o4, o5, e0, wr2, wl2, ba2, batch3)


def _gx_body(o1, o2, o3, o4, xct, bat, m, den, wg, bg, a2, ab2,
             sc_ref, gx_s):
    g = pl.program_id(0)

    @pl.when(g == 0)
    def _init():
        gx_s[...] = jnp.zeros((NITER, B, D), jnp.float32)

    outs = (o1[...], o2[...], o3[...], o4[...])
    ids = bat[0, 0]
    ohTf = (lax.broadcasted_iota(jnp.int32, (B, BE), 0)
            == ids[None, :]).astype(jnp.float32)
    for n in range(NITER):
        xcn = xct[n]
        mrow = jnp.dot(m[n], ohTf, preferred_element_type=jnp.float32)
        drow = jnp.dot(den[n], ohTf, preferred_element_type=jnp.float32)
        w = jnp.exp(xcn - mrow) / drow
        gx_s[n] = gx_s[n] + jnp.dot(ohTf, outs[n] * w[:, None],
                                    preferred_element_type=jnp.float32)

    @pl.when(g == NBLK_E - 1)
    def _fin():
        ss = []
        for n in range(NITER):
            gout = jnp.tanh(jnp.dot(gx_s[n], wg[...],
                                    preferred_element_type=jnp.float32) + bg[...])
            ss.append(jnp.sum(gout * a2[n][None, :], axis=1))
        s = jnp.stack(ss, axis=0) + ab2[...]
        smax = jnp.max(s, axis=0)
        e = jnp.exp(s - smax[None, :])
        sc_ref[...] = e / jnp.sum(e, axis=0)[None, :]


def _gx_scores(o1, o2, o3, o4, xct, batch3, m, den, W_gout, bg2, a2, ab2):
    blk = pl.BlockSpec((BE, D), lambda g: (g, 0))
    sml = pl.BlockSpec((NITER, B), lambda g: (0, 0))
    return pl.pallas_call(
        _gx_body,
        grid=(NBLK_E,),
        in_specs=[blk, blk, blk, blk,
                  pl.BlockSpec((NITER, BE), lambda g: (0, g)),
                  pl.BlockSpec((1, 1, BE), lambda g: (g, 0, 0)),
                  sml, sml,
                  pl.BlockSpec((D, D), lambda g: (0, 0)),
                  pl.BlockSpec((1, D), lambda g: (0, 0)),
                  pl.BlockSpec((NITER, D), lambda g: (0, 0)),
                  pl.BlockSpec((NITER, 1), lambda g: (0, 0))],
        out_specs=pl.BlockSpec((NITER, B), lambda g: (0, 0)),
        out_shape=jax.ShapeDtypeStruct((NITER, B), jnp.float32),
        scratch_shapes=[pltpu.VMEM((NITER, B, D), jnp.float32)],
    )(o1, o2, o3, o4, xct, batch3, m, den, W_gout, bg2, a2, ab2)


def _finpool_body(o1, o2, o3, o4, bat, sc, o_ref):
    outs = (o1[...], o2[...], o3[...], o4[...])
    ids = bat[0, 0]
    ohTf = (lax.broadcasted_iota(jnp.int32, (B, BE), 0)
            == ids[None, :]).astype(jnp.float32)
    acc = jnp.zeros((BE, D), jnp.float32)
    for n in range(NITER):
        w = jnp.dot(sc[n], ohTf, preferred_element_type=jnp.float32)
        acc = acc + outs[n] * w[:, None]
    o_ref[...] = acc


def _finpool(o1, o2, o3, o4, batch3, scores):
    blk = pl.BlockSpec((BE, D), lambda g: (g, 0))
    return pl.pallas_call(
        _finpool_body,
        grid=(NBLK_E,),
        in_specs=[blk, blk, blk, blk,
                  pl.BlockSpec((1, 1, BE), lambda g: (g, 0, 0)),
                  pl.BlockSpec((NITER, B), lambda g: (0, 0))],
        out_specs=blk,
        out_shape=jax.ShapeDtypeStruct((E, D), jnp.float32),
    )(o1, o2, o3, o4, batch3, scores)


@functools.cache
def _node_scatter_fn():
    return functools.partial(
        pl.kernel,
        out_type=(jax.ShapeDtypeStruct((N, D), jnp.float32),
                  jax.ShapeDtypeStruct((N, D), jnp.float32)),
        mesh=_mesh(),
        compiler_params=pltpu.CompilerParams(needs_layout_passes=False),
        scratch_types=[
            pltpu.VMEM_SHARED((N, D), jnp.float32),
            pltpu.VMEM((K,), jnp.int32),
            pltpu.VMEM((K, D), jnp.float32),
        ],
    )(_node_scatter_body)


def _node_scatter_body(h_hbm, fin_hbm, dst_hbm, p0_hbm, p1_hbm,
                       shared, idx_stage, rows):
    cid = lax.axis_index("c")
    sid = lax.axis_index("s")
    nrows = 624
    ntail = N - NS * nrows

    pltpu.sync_copy(h_hbm.at[pl.ds(sid * nrows, nrows)],
                    shared.at[pl.ds(sid * nrows, nrows)])

    @pl.when(sid == NS - 1)
    def _init_tail():
        pltpu.sync_copy(h_hbm.at[pl.ds(NS * nrows, ntail)],
                        shared.at[pl.ds(NS * nrows, ntail)])

    plsc.subcore_barrier()

    eh = E // NC
    nbtot = eh // K
    nfull = nbtot // NS
    nb = jnp.where(sid < (nbtot - nfull * NS), nfull + 1, nfull)

    def body(i, _):
        base = cid * eh + (i * NS + sid) * K
        pltpu.sync_copy(dst_hbm.at[pl.ds(base, K)], idx_stage)
        pltpu.sync_copy(fin_hbm.at[pl.ds(base, K)], rows)
        pltpu.sync_copy(rows, shared.at[idx_stage], add=True)
        return ()

    lax.fori_loop(0, nb, body, (), unroll=False)
    plsc.subcore_barrier()

    @pl.when(cid == 0)
    def _w0():
        pltpu.sync_copy(shared.at[pl.ds(sid * nrows, nrows)],
                        p0_hbm.at[pl.ds(sid * nrows, nrows)])

        @pl.when(sid == NS - 1)
        def _w0t():
            pltpu.sync_copy(shared.at[pl.ds(NS * nrows, ntail)],
                            p0_hbm.at[pl.ds(NS * nrows, ntail)])

    @pl.when(cid == 1)
    def _w1():
        pltpu.sync_copy(shared.at[pl.ds(sid * nrows, nrows)],
                        p1_hbm.at[pl.ds(sid * nrows, nrows)])

        @pl.when(sid == NS - 1)
        def _w1t():
            pltpu.sync_copy(shared.at[pl.ds(NS * nrows, ntail)],
                            p1_hbm.at[pl.ds(NS * nrows, ntail)])


def _final_body(p0, p1, hr, wl_ref, bl_ref, o_ref):
    xo = p0[...] + p1[...] - hr[...]
    o_ref[...] = jnp.dot(xo, wl_ref[...],
                         preferred_element_type=jnp.float32) + bl_ref[...]


def _final(p0, p1, h, W_lb, bl2):
    return pl.pallas_call(
        _final_body,
        grid=(NBLK_N,),
        in_specs=[pl.BlockSpec((BN, D), lambda g: (g, 0))] * 3 +
                 [pl.BlockSpec((D, D), lambda g: (0, 0)),
                  pl.BlockSpec((1, D), lambda g: (0, 0))],
        out_specs=pl.BlockSpec((BN, D), lambda g: (g, 0)),
        out_shape=jax.ShapeDtypeStruct((N, D), jnp.float32),
    )(p0, p1, h, W_lb, bl2)


def kernel(x, edge_attr, edge_index, line_graph_edge_index, edge_index_batch,
           W_mlp, b_mlp, W_u, W_v, W_edge, W_att_root, W_att_rel, b_att, a,
           W_gout, b_gout, a_bias, W_lb, b_lb):
    src = edge_index[0].astype(jnp.int32)
    dst = edge_index[1].astype(jnp.int32)
    lgs = line_graph_edge_index[0].astype(jnp.int32)
    lgd = line_graph_edge_index[1].astype(jnp.int32)
    batch3 = edge_index_batch.astype(jnp.int32).reshape(NBLK_E, 1, BE)

    h, eu3, ev3 = _prep_node(x, W_mlp, b_mlp.reshape(1, D), W_u, W_v)
    euv3 = _prep_edge(edge_attr, W_edge)
    geu, gev = _gather2_fn()(eu3, ev3, src, dst)
    e0 = _add3(geu, gev, euv3)

    outs = [e0]
    for _ in range(NITER + 1):
        outs.append(_lg_pass_fn()(outs[-1], e0, lgs, lgd))

    wr2 = W_att_root.reshape(1, D)
    wl2 = W_att_rel.reshape(1, D)
    ba2 = b_att.reshape(1, 1)
    xct, m, den = _xcstats(outs[1], outs[2], outs[3], outs[4], outs[5],
                           e0, wr2, wl2, ba2, batch3)

    a2 = jnp.transpose(a[0])
    ab2 = a_bias.reshape(NITER, 1)
    scores = _gx_scores(outs[1], outs[2], outs[3], outs[4], xct, batch3,
                        m, den, W_gout, b_gout.reshape(1, D), a2, ab2)

    out_fin = _finpool(outs[1], outs[2], outs[3], outs[4], batch3, scores)

    p0, p1 = _node_scatter_fn()(h, out_fin, dst)
    return _final(p0, p1, h, W_lb, b_lb.reshape(1, D))


# speedup vs baseline: 2.3541x; 1.1062x over previous
import functools

import jax
import jax.numpy as jnp
from jax import lax
from jax.experimental import pallas as pl
from jax.experimental.pallas import tpu as pltpu
from jax.experimental.pallas import tpu_sc as plsc

N = 10000
E = 160000
L = 320000
B = 256
D = 128
NITER = 4

NC = 2
NS = 16
NW = NC * NS

BE = 1280
NBLK_E = E // BE
BN = 1000
NBLK_N = N // BN

K = 128
KL = 96
CH = 10000
NCHUNK = E // CH
NROUND = NCHUNK // NC
LT = L // NS
LTP = LT + 160
PIECE = 2000

@functools.cache
def _mesh():
    return plsc.VectorSubcoreMesh(core_axis_name="c", subcore_axis_name="s",
                                  num_cores=NC, num_subcores=NS)

_NEG = -1e30


def _prep_node_body(x_ref, wm_ref, bm_ref, wu_ref, wv_ref, h_ref, eu_ref, ev_ref):
    h = jnp.dot(x_ref[...], wm_ref[...], preferred_element_type=jnp.float32)
    h = h + bm_ref[...]
    h_ref[...] = h
    third = jnp.float32(1.0 / 3.0)
    eu_ref[...] = jnp.dot(h, wu_ref[...], preferred_element_type=jnp.float32) * third
    ev_ref[...] = jnp.dot(h, wv_ref[...], preferred_element_type=jnp.float32) * third


def _prep_node(x, W_mlp, b_mlp2, W_u, W_v):
    return pl.pallas_call(
        _prep_node_body,
        grid=(NBLK_N,),
        in_specs=[
            pl.BlockSpec((BN, D), lambda g: (g, 0)),
            pl.BlockSpec((D, D), lambda g: (0, 0)),
            pl.BlockSpec((1, D), lambda g: (0, 0)),
            pl.BlockSpec((D, D), lambda g: (0, 0)),
            pl.BlockSpec((D, D), lambda g: (0, 0)),
        ],
        out_specs=[
            pl.BlockSpec((BN, D), lambda g: (g, 0)),
            pl.BlockSpec((BN, D), lambda g: (g, 0)),
            pl.BlockSpec((BN, D), lambda g: (g, 0)),
        ],
        out_shape=[jax.ShapeDtypeStruct((N, D), jnp.float32)] * 3,
    )(x, W_mlp, b_mlp2, W_u, W_v)


def _prep_edge_body(ea_ref, we_ref, o_ref):
    o_ref[...] = jnp.dot(ea_ref[...], we_ref[...],
                         preferred_element_type=jnp.float32) * jnp.float32(1.0 / 3.0)


def _prep_edge(edge_attr, W_edge):
    ed = edge_attr.shape[1]
    return pl.pallas_call(
        _prep_edge_body,
        grid=(NBLK_E,),
        in_specs=[
            pl.BlockSpec((BE, ed), lambda g: (g, 0)),
            pl.BlockSpec((ed, D), lambda g: (0, 0)),
        ],
        out_specs=pl.BlockSpec((BE, D), lambda g: (g, 0)),
        out_shape=jax.ShapeDtypeStruct((E, D), jnp.float32),
    )(edge_attr, W_edge)


@functools.cache
def _gather2_fn():
    return functools.partial(
        pl.kernel,
        out_type=(jax.ShapeDtypeStruct((E, D), jnp.float32),
                  jax.ShapeDtypeStruct((E, D), jnp.float32)),
        mesh=_mesh(),
        compiler_params=pltpu.CompilerParams(needs_layout_passes=False),
        scratch_types=[
            pltpu.VMEM((K,), jnp.int32),
            pltpu.VMEM((K,), jnp.int32),
            pltpu.VMEM((K, D), jnp.float32),
            pltpu.VMEM((K, D), jnp.float32),
            pltpu.SemaphoreType.DMA,
            pltpu.SemaphoreType.DMA,
        ],
    )(_gather2_body)


def _gather2_body(eu_hbm, ev_hbm, src_hbm, dst_hbm, geu_hbm, gev_hbm,
                  src_v, dst_v, bufa, bufb, sema, semb):
    wid = lax.axis_index("s") * NC + lax.axis_index("c")
    nbtot = E // K
    nfull = nbtot // NW
    nb = jnp.where(wid < (nbtot - nfull * NW), nfull + 1, nfull)

    def body(i, _):
        base = (i * NW + wid) * K
        pltpu.sync_copy(src_hbm.at[pl.ds(base, K)], src_v)
        pltpu.sync_copy(dst_hbm.at[pl.ds(base, K)], dst_v)
        ca = pltpu.async_copy(eu_hbm.at[src_v], bufa, sema)
        cb = pltpu.async_copy(ev_hbm.at[dst_v], bufb, semb)
        ca.wait()
        cb.wait()
        pltpu.sync_copy(bufa, geu_hbm.at[pl.ds(base, K)])
        pltpu.sync_copy(bufb, gev_hbm.at[pl.ds(base, K)])
        return ()

    lax.fori_loop(0, nb, body, (), unroll=False)


def _add3_body(a_ref, b_ref, c_ref, o_ref):
    o_ref[...] = a_ref[...] + b_ref[...] + c_ref[...]


def _add3(a, b, c):
    return pl.pallas_call(
        _add3_body,
        grid=(NBLK_E,),
        in_specs=[pl.BlockSpec((BE, D), lambda g: (g, 0))] * 3,
        out_specs=pl.BlockSpec((BE, D), lambda g: (g, 0)),
        out_shape=jax.ShapeDtypeStruct((E, D), jnp.float32),
    )(a, b, c)


@functools.cache
def _lg_pass_fn():
    return functools.partial(
        pl.kernel,
        out_type=jax.ShapeDtypeStruct((E, D), jnp.float32),
        mesh=_mesh(),
        compiler_params=pltpu.CompilerParams(needs_layout_passes=False),
        scratch_types=[
            pltpu.VMEM_SHARED((CH + 8, D), jnp.float32),
            pltpu.VMEM((PIECE,), jnp.int32),
            pltpu.VMEM((PIECE,), jnp.int32),
            pltpu.VMEM((LTP,), jnp.int32),
            pltpu.VMEM((KL,), jnp.int32),
            pltpu.VMEM((KL,), jnp.int32),
            pltpu.VMEM((KL,), jnp.int32),
            pltpu.VMEM((KL,), jnp.int32),
            pltpu.VMEM((KL, D), jnp.float32),
            pltpu.VMEM((KL, D), jnp.float32),
            pltpu.SemaphoreType.DMA,
            pltpu.SemaphoreType.DMA,
            pltpu.SemaphoreType.DMA,
        ],
    )(_lg_pass_body)


_SRC_BITS = 18
_PADW = ((CH << _SRC_BITS) & 0xFFFFFFFF) - (1 << 32)


def _lg_pass_body(table_hbm, init_hbm, lgs_hbm, lgd_hbm, out_hbm,
                  shared, dpiece, spiece, pbuf, rel0, src0, rel1, src1,
                  rows0, rows1, semi, semg0, semg1):
    cid = lax.axis_index("c")
    sid = lax.axis_index("s")
    trows = 624
    ttail = CH - NS * trows

    def round_body(r, _):
        base = (r * NC + cid) * CH

        pltpu.async_copy(init_hbm.at[pl.ds(base + sid * trows, trows)],
                         shared.at[pl.ds(sid * trows, trows)], semi)

        @pl.when(sid == NS - 1)
        def _init_tail():
            pltpu.async_copy(init_hbm.at[pl.ds(base + NS * trows, ttail)],
                             shared.at[pl.ds(NS * trows, ttail)], semi)

        def piece_body(p, cnt):
            off = sid * LT + p * PIECE
            pltpu.sync_copy(lgd_hbm.at[pl.ds(off, PIECE)], dpiece)
            pltpu.sync_copy(lgs_hbm.at[pl.ds(off, PIECE)], spiece)

            def scan_body(j, cnt2):
                d16 = dpiece[pl.ds(j * 16, 16)]
                s16 = spiece[pl.ds(j * 16, 16)]
                rel = d16 - base
                msk = (rel >= 0) & (rel < CH)
                packed = jnp.where(msk, (rel << _SRC_BITS) | s16,
                                   jnp.int32(_PADW))
                keys = jnp.where(msk, jnp.int32(0), jnp.int32(1))
                _, sv = plsc.sort_key_val(keys, packed)
                pbuf[pl.ds(cnt2, 16)] = sv
                return cnt2 + plsc.all_reduce_population_count(msk)[0]

            return lax.fori_loop(0, PIECE // 16, scan_body, cnt,
                                 unroll=False)

        cnt = lax.fori_loop(0, LT // PIECE, piece_body, jnp.int32(0),
                            unroll=False)

        padv = jnp.full((16,), jnp.int32(_PADW), jnp.int32)
        for t in range(KL // 16):
            pbuf[pl.ds(cnt + t * 16, 16)] = padv
        nb = (cnt + (KL - 1)) // KL

        pltpu.make_async_copy(init_hbm.at[pl.ds(base + sid * trows, trows)],
                              shared.at[pl.ds(sid * trows, trows)],
                              semi).wait()

        @pl.when(sid == NS - 1)
        def _init_tail_wait():
            pltpu.make_async_copy(
                init_hbm.at[pl.ds(base + NS * trows, ttail)],
                shared.at[pl.ds(NS * trows, ttail)], semi).wait()

        plsc.subcore_barrier()

        bufs = ((rel0, src0, rows0, semg0), (rel1, src1, rows1, semg1))

        def drain_pair(i2, _):
            b0 = i2 * 2
            for j in range(2):
                bi = b0 + j
                rl, sr, rw, sg = bufs[j]

                @pl.when(bi < nb)
                def _issue(bi=bi, rl=rl, sr=sr, rw=rw, sg=sg):
                    for t in range(KL // 16):
                        pk = pbuf[pl.ds(bi * KL + t * 16, 16)]
                        rl[pl.ds(t * 16, 16)] = lax.shift_right_logical(
                            pk, _SRC_BITS)
                        sr[pl.ds(t * 16, 16)] = pk & ((1 << _SRC_BITS) - 1)
                    pltpu.async_copy(table_hbm.at[sr], rw, sg)

            for j in range(2):
                bi = b0 + j
                rl, sr, rw, sg = bufs[j]

                @pl.when(bi < nb)
                def _drain(bi=bi, rl=rl, sr=sr, rw=rw, sg=sg):
                    pltpu.make_async_copy(table_hbm.at[sr], rw, sg).wait()
                    pltpu.sync_copy(rw, shared.at[rl], add=True)

            return ()

        lax.fori_loop(0, (nb + 1) // 2, drain_pair, (), unroll=False)
        plsc.subcore_barrier()

        pltpu.sync_copy(shared.at[pl.ds(sid * trows, trows)],
                        out_hbm.at[pl.ds(base + sid * trows, trows)])

        @pl.when(sid == NS - 1)
        def _out_tail():
            pltpu.sync_copy(shared.at[pl.ds(NS * trows, ttail)],
                            out_hbm.at[pl.ds(base + NS * trows, ttail)])

        return ()

    lax.fori_loop(0, NROUND, round_body, (), unroll=False)


def _xcstats_body(o1, o2, o3, o4, o5, e0r, wr, wl, ba, bat,
                  xct_ref, m_ref, den_ref, m_s, den_s):
    g = pl.program_id(0)

    @pl.when(g == 0)
    def _init():
        m_s[...] = jnp.full((NITER, B), _NEG, jnp.float32)
        den_s[...] = jnp.zeros((NITER, B), jnp.float32)

    outs = (o1[...], o2[...], o3[...], o4[...], o5[...])
    e0b = e0r[...]
    wrv = wr[...]
    wlv = wl[...]
    bav = ba[0, 0]
    ids = bat[0, 0]
    ohT = lax.broadcasted_iota(jnp.int32, (B, BE), 0) == ids[None, :]
    ohTf = ohT.astype(jnp.float32)

    xcs = []
    for n in range(NITER):
        xcn = (jnp.sum(outs[n] * wrv, axis=1)
               + jnp.sum((outs[n + 1] - e0b) * wlv, axis=1) + bav)
        xcs.append(xcn)
        contrib = jnp.max(jnp.where(ohT, xcn[None, :], _NEG), axis=1)
        mold = m_s[n]
        mnew = jnp.maximum(mold, contrib)
        mrow = jnp.dot(mnew, ohTf, preferred_element_type=jnp.float32)
        ex = jnp.exp(xcn - mrow)
        dc = jnp.dot(ohTf, ex, preferred_element_type=jnp.float32)
        den_s[n] = den_s[n] * jnp.exp(mold - mnew) + dc
        m_s[n] = mnew

    xct_ref[...] = jnp.stack(xcs, axis=0)

    @pl.when(g == NBLK_E - 1)
    def _fin():
        m_ref[...] = m_s[...]
        den_ref[...] = den_s[...]


def _xcstats(o1, o2, o3, o4, o5, e0, wr2, wl2, ba2, batch3):
    blk = pl.BlockSpec((BE, D), lambda g: (g, 0))
    return pl.pallas_call(
        _xcstats_body,
        grid=(NBLK_E,),
        in_specs=[blk, blk, blk, blk, blk, blk,
                  pl.BlockSpec((1, D), lambda g: (0, 0)),
                  pl.BlockSpec((1, D), lambda g: (0, 0)),
                  pl.BlockSpec((1, 1), lambda g: (0, 0)),
                  pl.BlockSpec((1, 1, BE), lambda g: (g, 0, 0))],
        out_specs=[pl.BlockSpec((NITER, BE), lambda g: (0, g)),
                   pl.BlockSpec((NITER, B), lambda g: (0, 0)),
                   pl.BlockSpec((NITER, B), lambda g: (0, 0))],
        out_shape=[jax.ShapeDtypeStruct((NITER, E), jnp.float32),
                   jax.ShapeDtypeStruct((NITER, B), jnp.float32),
                   jax.ShapeDtypeStruct((NITER, B), jnp.float32)],
        scratch_shapes=[pltpu.VMEM((NITER, B), jnp.float32),
                        pltpu.VMEM((NITER, B), jnp.float32)],
    )(o1, o2, o3, o4, o5, e0, wr2, wl2, ba2, batch3)


def _gx_body(o1, o2, o3, o4, xct, bat, m, den, wg, bg, a2, ab2,
             sc_ref, gx_s):
    g = pl.program_id(0)

    @pl.when(g == 0)
    def _init():
        gx_s[...] = jnp.zeros((NITER, B, D), jnp.float32)

    outs = (o1[...], o2[...], o3[...], o4[...])
    ids = bat[0, 0]
    ohTf = (lax.broadcasted_iota(jnp.int32, (B, BE), 0)
            == ids[None, :]).astype(jnp.float32)
    for n in range(NITER):
        xcn = xct[n]
        mrow = jnp.dot(m[n], ohTf, preferred_element_type=jnp.float32)
        drow = jnp.dot(den[n], ohTf, preferred_element_type=jnp.float32)
        w = jnp.exp(xcn - mrow) / drow
        gx_s[n] = gx_s[n] + jnp.dot(ohTf, outs[n] * w[:, None],
                                    preferred_element_type=jnp.float32)

    @pl.when(g == NBLK_E - 1)
    def _fin():
        ss = []
        for n in range(NITER):
            gout = jnp.tanh(jnp.dot(gx_s[n], wg[...],
                                    preferred_element_type=jnp.float32) + bg[...])
            ss.append(jnp.sum(gout * a2[n][None, :], axis=1))
        s = jnp.stack(ss, axis=0) + ab2[...]
        smax = jnp.max(s, axis=0)
        e = jnp.exp(s - smax[None, :])
        sc_ref[...] = e / jnp.sum(e, axis=0)[None, :]


def _gx_scores(o1, o2, o3, o4, xct, batch3, m, den, W_gout, bg2, a2, ab2):
    blk = pl.BlockSpec((BE, D), lambda g: (g, 0))
    sml = pl.BlockSpec((NITER, B), lambda g: (0, 0))
    return pl.pallas_call(
        _gx_body,
        grid=(NBLK_E,),
        in_specs=[blk, blk, blk, blk,
                  pl.BlockSpec((NITER, BE), lambda g: (0, g)),
                  pl.BlockSpec((1, 1, BE), lambda g: (g, 0, 0)),
                  sml, sml,
                  pl.BlockSpec((D, D), lambda g: (0, 0)),
                  pl.BlockSpec((1, D), lambda g: (0, 0)),
                  pl.BlockSpec((NITER, D), lambda g: (0, 0)),
                  pl.BlockSpec((NITER, 1), lambda g: (0, 0))],
        out_specs=pl.BlockSpec((NITER, B), lambda g: (0, 0)),
        out_shape=jax.ShapeDtypeStruct((NITER, B), jnp.float32),
        scratch_shapes=[pltpu.VMEM((NITER, B, D), jnp.float32)],
    )(o1, o2, o3, o4, xct, batch3, m, den, W_gout, bg2, a2, ab2)


def _finpool_body(o1, o2, o3, o4, bat, sc, o_ref):
    outs = (o1[...], o2[...], o3[...], o4[...])
    ids = bat[0, 0]
    ohTf = (lax.broadcasted_iota(jnp.int32, (B, BE), 0)
            == ids[None, :]).astype(jnp.float32)
    acc = jnp.zeros((BE, D), jnp.float32)
    for n in range(NITER):
        w = jnp.dot(sc[n], ohTf, preferred_element_type=jnp.float32)
        acc = acc + outs[n] * w[:, None]
    o_ref[...] = acc


def _finpool(o1, o2, o3, o4, batch3, scores):
    blk = pl.BlockSpec((BE, D), lambda g: (g, 0))
    return pl.pallas_call(
        _finpool_body,
        grid=(NBLK_E,),
        in_specs=[blk, blk, blk, blk,
                  pl.BlockSpec((1, 1, BE), lambda g: (g, 0, 0)),
                  pl.BlockSpec((NITER, B), lambda g: (0, 0))],
        out_specs=blk,
        out_shape=jax.ShapeDtypeStruct((E, D), jnp.float32),
    )(o1, o2, o3, o4, batch3, scores)


@functools.cache
def _node_scatter_fn():
    return functools.partial(
        pl.kernel,
        out_type=(jax.ShapeDtypeStruct((N, D), jnp.float32),
                  jax.ShapeDtypeStruct((N, D), jnp.float32)),
        mesh=_mesh(),
        compiler_params=pltpu.CompilerParams(needs_layout_passes=False),
        scratch_types=[
            pltpu.VMEM_SHARED((N, D), jnp.float32),
            pltpu.VMEM((K,), jnp.int32),
            pltpu.VMEM((K, D), jnp.float32),
        ],
    )(_node_scatter_body)


def _node_scatter_body(h_hbm, fin_hbm, dst_hbm, p0_hbm, p1_hbm,
                       shared, idx_stage, rows):
    cid = lax.axis_index("c")
    sid = lax.axis_index("s")
    nrows = 624
    ntail = N - NS * nrows

    pltpu.sync_copy(h_hbm.at[pl.ds(sid * nrows, nrows)],
                    shared.at[pl.ds(sid * nrows, nrows)])

    @pl.when(sid == NS - 1)
    def _init_tail():
        pltpu.sync_copy(h_hbm.at[pl.ds(NS * nrows, ntail)],
                        shared.at[pl.ds(NS * nrows, ntail)])

    plsc.subcore_barrier()

    eh = E // NC
    nbtot = eh // K
    nfull = nbtot // NS
    nb = jnp.where(sid < (nbtot - nfull * NS), nfull + 1, nfull)

    def body(i, _):
        base = cid * eh + (i * NS + sid) * K
        pltpu.sync_copy(dst_hbm.at[pl.ds(base, K)], idx_stage)
        pltpu.sync_copy(fin_hbm.at[pl.ds(base, K)], rows)
        pltpu.sync_copy(rows, shared.at[idx_stage], add=True)
        return ()

    lax.fori_loop(0, nb, body, (), unroll=False)
    plsc.subcore_barrier()

    @pl.when(cid == 0)
    def _w0():
        pltpu.sync_copy(shared.at[pl.ds(sid * nrows, nrows)],
                        p0_hbm.at[pl.ds(sid * nrows, nrows)])

        @pl.when(sid == NS - 1)
        def _w0t():
            pltpu.sync_copy(shared.at[pl.ds(NS * nrows, ntail)],
                            p0_hbm.at[pl.ds(NS * nrows, ntail)])

    @pl.when(cid == 1)
    def _w1():
        pltpu.sync_copy(shared.at[pl.ds(sid * nrows, nrows)],
                        p1_hbm.at[pl.ds(sid * nrows, nrows)])

        @pl.when(sid == NS - 1)
        def _w1t():
            pltpu.sync_copy(shared.at[pl.ds(NS * nrows, ntail)],
                            p1_hbm.at[pl.ds(NS * nrows, ntail)])


def _final_body(p0, p1, hr, wl_ref, bl_ref, o_ref):
    xo = p0[...] + p1[...] - hr[...]
    o_ref[...] = jnp.dot(xo, wl_ref[...],
                         preferred_element_type=jnp.float32) + bl_ref[...]


def _final(p0, p1, h, W_lb, bl2):
    return pl.pallas_call(
        _final_body,
        grid=(NBLK_N,),
        in_specs=[pl.BlockSpec((BN, D), lambda g: (g, 0))] * 3 +
                 [pl.BlockSpec((D, D), lambda g: (0, 0)),
                  pl.BlockSpec((1, D), lambda g: (0, 0))],
        out_specs=pl.BlockSpec((BN, D), lambda g: (g, 0)),
        out_shape=jax.ShapeDtypeStruct((N, D), jnp.float32),
    )(p0, p1, h, W_lb, bl2)


def kernel(x, edge_attr, edge_index, line_graph_edge_index, edge_index_batch,
           W_mlp, b_mlp, W_u, W_v, W_edge, W_att_root, W_att_rel, b_att, a,
           W_gout, b_gout, a_bias, W_lb, b_lb):
    src = edge_index[0].astype(jnp.int32)
    dst = edge_index[1].astype(jnp.int32)
    lgs = line_graph_edge_index[0].astype(jnp.int32)
    lgd = line_graph_edge_index[1].astype(jnp.int32)
    batch3 = edge_index_batch.astype(jnp.int32).reshape(NBLK_E, 1, BE)

    h, eu3, ev3 = _prep_node(x, W_mlp, b_mlp.reshape(1, D), W_u, W_v)
    euv3 = _prep_edge(edge_attr, W_edge)
    geu, gev = _gather2_fn()(eu3, ev3, src, dst)
    e0 = _add3(geu, gev, euv3)

    outs = [e0]
    for _ in range(NITER + 1):
        outs.append(_lg_pass_fn()(outs[-1], e0, lgs, lgd))

    wr2 = W_att_root.reshape(1, D)
    wl2 = W_att_rel.reshape(1, D)
    ba2 = b_att.reshape(1, 1)
    xct, m, den = _xcstats(outs[1], outs[2], outs[3], outs[4], outs[5],
                           e0, wr2, wl2, ba2, batch3)

    a2 = jnp.transpose(a[0])
    ab2 = a_bias.reshape(NITER, 1)
    scores = _gx_scores(outs[1], outs[2], outs[3], outs[4], xct, batch3,
                        m, den, W_gout, b_gout.reshape(1, D), a2, ab2)

    out_fin = _finpool(outs[1], outs[2], outs[3], outs[4], batch3, scores)

    p0, p1 = _node_scatter_fn()(h, out_fin, dst)
    return _final(p0, p1, h, W_lb, b_lb.reshape(1, D))


# speedup vs baseline: 2.5109x; 1.0666x over previous
import functools

import jax
import jax.numpy as jnp
from jax import lax
from jax.experimental import pallas as pl
from jax.experimental.pallas import tpu as pltpu
from jax.experimental.pallas import tpu_sc as plsc

N = 10000
E = 160000
L = 320000
B = 256
D = 128
NITER = 4

NC = 2
NS = 16
NW = NC * NS

BE = 1280
NBLK_E = E // BE
BN = 1000
NBLK_N = N // BN

K = 128
KL = 96
CH = 10000
NCHUNK = E // CH
NROUND = NCHUNK // NC
LT = L // NS
LTP = LT + 160
PIECE = 2000

@functools.cache
def _mesh():
    return plsc.VectorSubcoreMesh(core_axis_name="c", subcore_axis_name="s",
                                  num_cores=NC, num_subcores=NS)

_NEG = -1e30


def _prep_node_body(x_ref, wm_ref, bm_ref, wu_ref, wv_ref, h_ref, eu_ref, ev_ref):
    h = jnp.dot(x_ref[...], wm_ref[...], preferred_element_type=jnp.float32)
    h = h + bm_ref[...]
    h_ref[...] = h
    third = jnp.float32(1.0 / 3.0)
    eu_ref[...] = jnp.dot(h, wu_ref[...], preferred_element_type=jnp.float32) * third
    ev_ref[...] = jnp.dot(h, wv_ref[...], preferred_element_type=jnp.float32) * third


def _prep_node(x, W_mlp, b_mlp2, W_u, W_v):
    return pl.pallas_call(
        _prep_node_body,
        grid=(NBLK_N,),
        in_specs=[
            pl.BlockSpec((BN, D), lambda g: (g, 0)),
            pl.BlockSpec((D, D), lambda g: (0, 0)),
            pl.BlockSpec((1, D), lambda g: (0, 0)),
            pl.BlockSpec((D, D), lambda g: (0, 0)),
            pl.BlockSpec((D, D), lambda g: (0, 0)),
        ],
        out_specs=[
            pl.BlockSpec((BN, D), lambda g: (g, 0)),
            pl.BlockSpec((BN, D), lambda g: (g, 0)),
            pl.BlockSpec((BN, D), lambda g: (g, 0)),
        ],
        out_shape=[jax.ShapeDtypeStruct((N, D), jnp.float32)] * 3,
    )(x, W_mlp, b_mlp2, W_u, W_v)


def _prep_edge_body(ea_ref, we_ref, o_ref):
    o_ref[...] = jnp.dot(ea_ref[...], we_ref[...],
                         preferred_element_type=jnp.float32) * jnp.float32(1.0 / 3.0)


def _prep_edge(edge_attr, W_edge):
    ed = edge_attr.shape[1]
    return pl.pallas_call(
        _prep_edge_body,
        grid=(NBLK_E,),
        in_specs=[
            pl.BlockSpec((BE, ed), lambda g: (g, 0)),
            pl.BlockSpec((ed, D), lambda g: (0, 0)),
        ],
        out_specs=pl.BlockSpec((BE, D), lambda g: (g, 0)),
        out_shape=jax.ShapeDtypeStruct((E, D), jnp.float32),
    )(edge_attr, W_edge)


@functools.cache
def _gather2_fn():
    return functools.partial(
        pl.kernel,
        out_type=(jax.ShapeDtypeStruct((E, D), jnp.float32),
                  jax.ShapeDtypeStruct((E, D), jnp.float32)),
        mesh=_mesh(),
        compiler_params=pltpu.CompilerParams(needs_layout_passes=False),
        scratch_types=[
            pltpu.VMEM((K,), jnp.int32),
            pltpu.VMEM((K,), jnp.int32),
            pltpu.VMEM((K, D), jnp.float32),
            pltpu.VMEM((K, D), jnp.float32),
            pltpu.SemaphoreType.DMA,
            pltpu.SemaphoreType.DMA,
        ],
    )(_gather2_body)


def _gather2_body(eu_hbm, ev_hbm, src_hbm, dst_hbm, geu_hbm, gev_hbm,
                  src_v, dst_v, bufa, bufb, sema, semb):
    wid = lax.axis_index("s") * NC + lax.axis_index("c")
    nbtot = E // K
    nfull = nbtot // NW
    nb = jnp.where(wid < (nbtot - nfull * NW), nfull + 1, nfull)

    def body(i, _):
        base = (i * NW + wid) * K
        pltpu.sync_copy(src_hbm.at[pl.ds(base, K)], src_v)
        pltpu.sync_copy(dst_hbm.at[pl.ds(base, K)], dst_v)
        ca = pltpu.async_copy(eu_hbm.at[src_v], bufa, sema)
        cb = pltpu.async_copy(ev_hbm.at[dst_v], bufb, semb)
        ca.wait()
        cb.wait()
        pltpu.sync_copy(bufa, geu_hbm.at[pl.ds(base, K)])
        pltpu.sync_copy(bufb, gev_hbm.at[pl.ds(base, K)])
        return ()

    lax.fori_loop(0, nb, body, (), unroll=False)


def _add3_body(a_ref, b_ref, c_ref, o_ref):
    o_ref[...] = a_ref[...] + b_ref[...] + c_ref[...]


def _add3(a, b, c):
    return pl.pallas_call(
        _add3_body,
        grid=(NBLK_E,),
        in_specs=[pl.BlockSpec((BE, D), lambda g: (g, 0))] * 3,
        out_specs=pl.BlockSpec((BE, D), lambda g: (g, 0)),
        out_shape=jax.ShapeDtypeStruct((E, D), jnp.float32),
    )(a, b, c)


@functools.cache
def _lg_pass_fn():
    return functools.partial(
        pl.kernel,
        out_type=jax.ShapeDtypeStruct((E, D), jnp.float32),
        mesh=_mesh(),
        compiler_params=pltpu.CompilerParams(needs_layout_passes=False),
        scratch_types=[
            pltpu.VMEM_SHARED((CH + 8, D), jnp.float32),
            pltpu.VMEM((PIECE,), jnp.int32),
            pltpu.VMEM((PIECE,), jnp.int32),
            pltpu.VMEM((LTP,), jnp.int32),
            pltpu.VMEM((KL,), jnp.int32),
            pltpu.VMEM((KL,), jnp.int32),
            pltpu.VMEM((KL,), jnp.int32),
            pltpu.VMEM((KL,), jnp.int32),
            pltpu.VMEM((KL, D), jnp.float32),
            pltpu.VMEM((KL, D), jnp.float32),
            pltpu.SemaphoreType.DMA,
            pltpu.SemaphoreType.DMA,
            pltpu.SemaphoreType.DMA,
            pltpu.SemaphoreType.DMA,
            pltpu.SemaphoreType.DMA,
        ],
    )(_lg_pass_body)


_SRC_BITS = 18
_PADW = ((CH << _SRC_BITS) & 0xFFFFFFFF) - (1 << 32)


def _lg_pass_body(table_hbm, init_hbm, lgs_hbm, lgd_hbm, out_hbm,
                  shared, dpiece, spiece, pbuf, rel0, src0, rel1, src1,
                  rows0, rows1, semi, semg0, semg1, sems0, sems1):
    cid = lax.axis_index("c")
    sid = lax.axis_index("s")
    trows = 624
    ttail = CH - NS * trows

    def round_body(r, _):
        base = (r * NC + cid) * CH

        pltpu.async_copy(init_hbm.at[pl.ds(base + sid * trows, trows)],
                         shared.at[pl.ds(sid * trows, trows)], semi)

        @pl.when(sid == NS - 1)
        def _init_tail():
            pltpu.async_copy(init_hbm.at[pl.ds(base + NS * trows, ttail)],
                             shared.at[pl.ds(NS * trows, ttail)], semi)

        def piece_body(p, cnt):
            off = sid * LT + p * PIECE
            pltpu.sync_copy(lgd_hbm.at[pl.ds(off, PIECE)], dpiece)
            pltpu.sync_copy(lgs_hbm.at[pl.ds(off, PIECE)], spiece)

            def scan_body(j, cnt2):
                da = dpiece[pl.ds(j * 32, 16)]
                db = dpiece[pl.ds(j * 32 + 16, 16)]
                sa = spiece[pl.ds(j * 32, 16)]
                sb = spiece[pl.ds(j * 32 + 16, 16)]
                ra = da - base
                rb = db - base
                ma = (ra >= 0) & (ra < CH)
                mb = (rb >= 0) & (rb < CH)
                pa = jnp.where(ma, (ra << _SRC_BITS) | sa, jnp.int32(_PADW))
                pb = jnp.where(mb, (rb << _SRC_BITS) | sb, jnp.int32(_PADW))
                ka = jnp.where(ma, jnp.int32(0), jnp.int32(1))
                kb = jnp.where(mb, jnp.int32(0), jnp.int32(1))
                _, va = plsc.sort_key_val(ka, pa)
                _, vb = plsc.sort_key_val(kb, pb)
                ca = plsc.all_reduce_population_count(ma)[0]
                cb = plsc.all_reduce_population_count(mb)[0]
                pbuf[pl.ds(cnt2, 16)] = va
                pbuf[pl.ds(cnt2 + ca, 16)] = vb
                return cnt2 + ca + cb

            return lax.fori_loop(0, PIECE // 32, scan_body, cnt,
                                 unroll=False)

        cnt = lax.fori_loop(0, LT // PIECE, piece_body, jnp.int32(0),
                            unroll=False)

        padv = jnp.full((16,), jnp.int32(_PADW), jnp.int32)
        for t in range(KL // 16):
            pbuf[pl.ds(cnt + t * 16, 16)] = padv
        nb = (cnt + (KL - 1)) // KL

        pltpu.make_async_copy(init_hbm.at[pl.ds(base + sid * trows, trows)],
                              shared.at[pl.ds(sid * trows, trows)],
                              semi).wait()

        @pl.when(sid == NS - 1)
        def _init_tail_wait():
            pltpu.make_async_copy(
                init_hbm.at[pl.ds(base + NS * trows, ttail)],
                shared.at[pl.ds(NS * trows, ttail)], semi).wait()

        plsc.subcore_barrier()

        bufs = ((rel0, src0, rows0, semg0, sems0),
                (rel1, src1, rows1, semg1, sems1))

        def drain_pair(i2, _):
            b0 = i2 * 2
            for j in range(2):
                bi = b0 + j
                rl, sr, rw, sg, ss = bufs[j]

                @pl.when(bi < nb)
                def _issue(bi=bi, rl=rl, sr=sr, rw=rw, sg=sg, ss=ss):
                    @pl.when(bi >= 2)
                    def _reuse_wait():
                        pltpu.make_async_copy(rw, shared.at[rl], ss).wait()

                    for t in range(KL // 16):
                        pk = pbuf[pl.ds(bi * KL + t * 16, 16)]
                        rl[pl.ds(t * 16, 16)] = lax.shift_right_logical(
                            pk, _SRC_BITS)
                        sr[pl.ds(t * 16, 16)] = pk & ((1 << _SRC_BITS) - 1)
                    pltpu.async_copy(table_hbm.at[sr], rw, sg)

            for j in range(2):
                bi = b0 + j
                rl, sr, rw, sg, ss = bufs[j]

                @pl.when(bi < nb)
                def _drain(bi=bi, rl=rl, sr=sr, rw=rw, sg=sg, ss=ss):
                    pltpu.make_async_copy(table_hbm.at[sr], rw, sg).wait()
                    pltpu.async_copy(rw, shared.at[rl], ss, add=True)

            return ()

        lax.fori_loop(0, (nb + 1) // 2, drain_pair, (), unroll=False)

        for j in range(2):
            rl, sr, rw, sg, ss = bufs[j]

            @pl.when(nb > j)
            def _final_drain(rl=rl, rw=rw, ss=ss):
                pltpu.make_async_copy(rw, shared.at[rl], ss).wait()

        plsc.subcore_barrier()

        pltpu.sync_copy(shared.at[pl.ds(sid * trows, trows)],
                        out_hbm.at[pl.ds(base + sid * trows, trows)])

        @pl.when(sid == NS - 1)
        def _out_tail():
            pltpu.sync_copy(shared.at[pl.ds(NS * trows, ttail)],
                            out_hbm.at[pl.ds(base + NS * trows, ttail)])

        return ()

    lax.fori_loop(0, NROUND, round_body, (), unroll=False)


def _xcstats_body(o1, o2, o3, o4, o5, e0r, wr, wl, ba, bat,
                  xct_ref, m_ref, den_ref, m_s, den_s):
    g = pl.program_id(0)

    @pl.when(g == 0)
    def _init():
        m_s[...] = jnp.full((NITER, B), _NEG, jnp.float32)
        den_s[...] = jnp.zeros((NITER, B), jnp.float32)

    outs = (o1[...], o2[...], o3[...], o4[...], o5[...])
    e0b = e0r[...]
    wrv = wr[...]
    wlv = wl[...]
    bav = ba[0, 0]
    ids = bat[0, 0]
    ohT = lax.broadcasted_iota(jnp.int32, (B, BE), 0) == ids[None, :]
    ohTf = ohT.astype(jnp.float32)

    xcs = []
    for n in range(NITER):
        xcn = (jnp.sum(outs[n] * wrv, axis=1)
               + jnp.sum((outs[n + 1] - e0b) * wlv, axis=1) + bav)
        xcs.append(xcn)
        contrib = jnp.max(jnp.where(ohT, xcn[None, :], _NEG), axis=1)
        mold = m_s[n]
        mnew = jnp.maximum(mold, contrib)
        mrow = jnp.dot(mnew, ohTf, preferred_element_type=jnp.float32)
        ex = jnp.exp(xcn - mrow)
        dc = jnp.dot(ohTf, ex, preferred_element_type=jnp.float32)
        den_s[n] = den_s[n] * jnp.exp(mold - mnew) + dc
        m_s[n] = mnew

    xct_ref[...] = jnp.stack(xcs, axis=0)

    @pl.when(g == NBLK_E - 1)
    def _fin():
        m_ref[...] = m_s[...]
        den_ref[...] = den_s[...]


def _xcstats(o1, o2, o3, o4, o5, e0, wr2, wl2, ba2, batch3):
    blk = pl.BlockSpec((BE, D), lambda g: (g, 0))
    return pl.pallas_call(
        _xcstats_body,
        grid=(NBLK_E,),
        in_specs=[blk, blk, blk, blk, blk, blk,
                  pl.BlockSpec((1, D), lambda g: (0, 0)),
                  pl.BlockSpec((1, D), lambda g: (0, 0)),
                  pl.BlockSpec((1, 1), lambda g: (0, 0)),
                  pl.BlockSpec((1, 1, BE), lambda g: (g, 0, 0))],
        out_specs=[pl.BlockSpec((NITER, BE), lambda g: (0, g)),
                   pl.BlockSpec((NITER, B), lambda g: (0, 0)),
                   pl.BlockSpec((NITER, B), lambda g: (0, 0))],
        out_shape=[jax.ShapeDtypeStruct((NITER, E), jnp.float32),
                   jax.ShapeDtypeStruct((NITER, B), jnp.float32),
                   jax.ShapeDtypeStruct((NITER, B), jnp.float32)],
        scratch_shapes=[pltpu.VMEM((NITER, B), jnp.float32),
                        pltpu.VMEM((NITER, B), jnp.float32)],
    )(o1, o2, o3, o4, o5, e0, wr2, wl2, ba2, batch3)


def _gx_body(o1, o2, o3, o4, xct, bat, m, den, wg, bg, a2, ab2,
             sc_ref, gx_s):
    g = pl.program_id(0)

    @pl.when(g == 0)
    def _init():
        gx_s[...] = jnp.zeros((NITER, B, D), jnp.float32)

    outs = (o1[...], o2[...], o3[...], o4[...])
    ids = bat[0, 0]
    ohTf = (lax.broadcasted_iota(jnp.int32, (B, BE), 0)
            == ids[None, :]).astype(jnp.float32)
    for n in range(NITER):
        xcn = xct[n]
        mrow = jnp.dot(m[n], ohTf, preferred_element_type=jnp.float32)
        drow = jnp.dot(den[n], ohTf, preferred_element_type=jnp.float32)
        w = jnp.exp(xcn - mrow) / drow
        gx_s[n] = gx_s[n] + jnp.dot(ohTf, outs[n] * w[:, None],
                                    preferred_element_type=jnp.float32)

    @pl.when(g == NBLK_E - 1)
    def _fin():
        ss = []
        for n in range(NITER):
            gout = jnp.tanh(jnp.dot(gx_s[n], wg[...],
                                    preferred_element_type=jnp.float32) + bg[...])
            ss.append(jnp.sum(gout * a2[n][None, :], axis=1))
        s = jnp.stack(ss, axis=0) + ab2[...]
        smax = jnp.max(s, axis=0)
        e = jnp.exp(s - smax[None, :])
        sc_ref[...] = e / jnp.sum(e, axis=0)[None, :]


def _gx_scores(o1, o2, o3, o4, xct, batch3, m, den, W_gout, bg2, a2, ab2):
    blk = pl.BlockSpec((BE, D), lambda g: (g, 0))
    sml = pl.BlockSpec((NITER, B), lambda g: (0, 0))
    return pl.pallas_call(
        _gx_body,
        grid=(NBLK_E,),
        in_specs=[blk, blk, blk, blk,
                  pl.BlockSpec((NITER, BE), lambda g: (0, g)),
                  pl.BlockSpec((1, 1, BE), lambda g: (g, 0, 0)),
                  sml, sml,
                  pl.BlockSpec((D, D), lambda g: (0, 0)),
                  pl.BlockSpec((1, D), lambda g: (0, 0)),
                  pl.BlockSpec((NITER, D), lambda g: (0, 0)),
                  pl.BlockSpec((NITER, 1), lambda g: (0, 0))],
        out_specs=pl.BlockSpec((NITER, B), lambda g: (0, 0)),
        out_shape=jax.ShapeDtypeStruct((NITER, B), jnp.float32),
        scratch_shapes=[pltpu.VMEM((NITER, B, D), jnp.float32)],
    )(o1, o2, o3, o4, xct, batch3, m, den, W_gout, bg2, a2, ab2)


def _finpool_body(o1, o2, o3, o4, bat, sc, o_ref):
    outs = (o1[...], o2[...], o3[...], o4[...])
    ids = bat[0, 0]
    ohTf = (lax.broadcasted_iota(jnp.int32, (B, BE), 0)
            == ids[None, :]).astype(jnp.float32)
    acc = jnp.zeros((BE, D), jnp.float32)
    for n in range(NITER):
        w = jnp.dot(sc[n], ohTf, preferred_element_type=jnp.float32)
        acc = acc + outs[n] * w[:, None]
    o_ref[...] = acc


def _finpool(o1, o2, o3, o4, batch3, scores):
    blk = pl.BlockSpec((BE, D), lambda g: (g, 0))
    return pl.pallas_call(
        _finpool_body,
        grid=(NBLK_E,),
        in_specs=[blk, blk, blk, blk,
                  pl.BlockSpec((1, 1, BE), lambda g: (g, 0, 0)),
                  pl.BlockSpec((NITER, B), lambda g: (0, 0))],
        out_specs=blk,
        out_shape=jax.ShapeDtypeStruct((E, D), jnp.float32),
    )(o1, o2, o3, o4, batch3, scores)


@functools.cache
def _node_scatter_fn():
    return functools.partial(
        pl.kernel,
        out_type=(jax.ShapeDtypeStruct((N, D), jnp.float32),
                  jax.ShapeDtypeStruct((N, D), jnp.float32)),
        mesh=_mesh(),
        compiler_params=pltpu.CompilerParams(needs_layout_passes=False),
        scratch_types=[
            pltpu.VMEM_SHARED((N, D), jnp.float32),
            pltpu.VMEM((K,), jnp.int32),
            pltpu.VMEM((K, D), jnp.float32),
        ],
    )(_node_scatter_body)


def _node_scatter_body(h_hbm, fin_hbm, dst_hbm, p0_hbm, p1_hbm,
                       shared, idx_stage, rows):
    cid = lax.axis_index("c")
    sid = lax.axis_index("s")
    nrows = 624
    ntail = N - NS * nrows

    pltpu.sync_copy(h_hbm.at[pl.ds(sid * nrows, nrows)],
                    shared.at[pl.ds(sid * nrows, nrows)])

    @pl.when(sid == NS - 1)
    def _init_tail():
        pltpu.sync_copy(h_hbm.at[pl.ds(NS * nrows, ntail)],
                        shared.at[pl.ds(NS * nrows, ntail)])

    plsc.subcore_barrier()

    eh = E // NC
    nbtot = eh // K
    nfull = nbtot // NS
    nb = jnp.where(sid < (nbtot - nfull * NS), nfull + 1, nfull)

    def body(i, _):
        base = cid * eh + (i * NS + sid) * K
        pltpu.sync_copy(dst_hbm.at[pl.ds(base, K)], idx_stage)
        pltpu.sync_copy(fin_hbm.at[pl.ds(base, K)], rows)
        pltpu.sync_copy(rows, shared.at[idx_stage], add=True)
        return ()

    lax.fori_loop(0, nb, body, (), unroll=False)
    plsc.subcore_barrier()

    @pl.when(cid == 0)
    def _w0():
        pltpu.sync_copy(shared.at[pl.ds(sid * nrows, nrows)],
                        p0_hbm.at[pl.ds(sid * nrows, nrows)])

        @pl.when(sid == NS - 1)
        def _w0t():
            pltpu.sync_copy(shared.at[pl.ds(NS * nrows, ntail)],
                            p0_hbm.at[pl.ds(NS * nrows, ntail)])

    @pl.when(cid == 1)
    def _w1():
        pltpu.sync_copy(shared.at[pl.ds(sid * nrows, nrows)],
                        p1_hbm.at[pl.ds(sid * nrows, nrows)])

        @pl.when(sid == NS - 1)
        def _w1t():
            pltpu.sync_copy(shared.at[pl.ds(NS * nrows, ntail)],
                            p1_hbm.at[pl.ds(NS * nrows, ntail)])


def _final_body(p0, p1, hr, wl_ref, bl_ref, o_ref):
    xo = p0[...] + p1[...] - hr[...]
    o_ref[...] = jnp.dot(xo, wl_ref[...],
                         preferred_element_type=jnp.float32) + bl_ref[...]


def _final(p0, p1, h, W_lb, bl2):
    return pl.pallas_call(
        _final_body,
        grid=(NBLK_N,),
        in_specs=[pl.BlockSpec((BN, D), lambda g: (g, 0))] * 3 +
                 [pl.BlockSpec((D, D), lambda g: (0, 0)),
                  pl.BlockSpec((1, D), lambda g: (0, 0))],
        out_specs=pl.BlockSpec((BN, D), lambda g: (g, 0)),
        out_shape=jax.ShapeDtypeStruct((N, D), jnp.float32),
    )(p0, p1, h, W_lb, bl2)


def kernel(x, edge_attr, edge_index, line_graph_edge_index, edge_index_batch,
           W_mlp, b_mlp, W_u, W_v, W_edge, W_att_root, W_att_rel, b_att, a,
           W_gout, b_gout, a_bias, W_lb, b_lb):
    src = edge_index[0].astype(jnp.int32)
    dst = edge_index[1].astype(jnp.int32)
    lgs = line_graph_edge_index[0].astype(jnp.int32)
    lgd = line_graph_edge_index[1].astype(jnp.int32)
    batch3 = edge_index_batch.astype(jnp.int32).reshape(NBLK_E, 1, BE)

    h, eu3, ev3 = _prep_node(x, W_mlp, b_mlp.reshape(1, D), W_u, W_v)
    euv3 = _prep_edge(edge_attr, W_edge)
    geu, gev = _gather2_fn()(eu3, ev3, src, dst)
    e0 = _add3(geu, gev, euv3)

    outs = [e0]
    for _ in range(NITER + 1):
        outs.append(_lg_pass_fn()(outs[-1], e0, lgs, lgd))

    wr2 = W_att_root.reshape(1, D)
    wl2 = W_att_rel.reshape(1, D)
    ba2 = b_att.reshape(1, 1)
    xct, m, den = _xcstats(outs[1], outs[2], outs[3], outs[4], outs[5],
                           e0, wr2, wl2, ba2, batch3)

    a2 = jnp.transpose(a[0])
    ab2 = a_bias.reshape(NITER, 1)
    scores = _gx_scores(outs[1], outs[2], outs[3], outs[4], xct, batch3,
                        m, den, W_gout, b_gout.reshape(1, D), a2, ab2)

    out_fin = _finpool(outs[1], outs[2], outs[3], outs[4], batch3, scores)

    p0, p1 = _node_scatter_fn()(h, out_fin, dst)
    return _final(p0, p1, h, W_lb, b_lb.reshape(1, D))


# speedup vs baseline: 2.6718x; 1.0641x over previous
import functools

import jax
import jax.numpy as jnp
from jax import lax
from jax.experimental import pallas as pl
from jax.experimental.pallas import tpu as pltpu
from jax.experimental.pallas import tpu_sc as plsc

N = 10000
E = 160000
L = 320000
B = 256
D = 128
NITER = 4

NC = 2
NS = 16
NW = NC * NS

BE = 1280
NBLK_E = E // BE
BN = 1000
NBLK_N = N // BN

K = 128
KL = 96
CH = 10000
NCHUNK = E // CH
NROUND = NCHUNK // NC
LT = L // NS
LTP = LT + 160
PIECE = 2000

@functools.cache
def _mesh():
    return plsc.VectorSubcoreMesh(core_axis_name="c", subcore_axis_name="s",
                                  num_cores=NC, num_subcores=NS)

_NEG = -1e30


def _prep_node_body(x_ref, wm_ref, bm_ref, wu_ref, wv_ref, h_ref, eu_ref, ev_ref):
    h = jnp.dot(x_ref[...], wm_ref[...], preferred_element_type=jnp.float32)
    h = h + bm_ref[...]
    h_ref[...] = h
    third = jnp.float32(1.0 / 3.0)
    eu_ref[...] = jnp.dot(h, wu_ref[...], preferred_element_type=jnp.float32) * third
    ev_ref[...] = jnp.dot(h, wv_ref[...], preferred_element_type=jnp.float32) * third


def _prep_node(x, W_mlp, b_mlp2, W_u, W_v):
    return pl.pallas_call(
        _prep_node_body,
        grid=(NBLK_N,),
        in_specs=[
            pl.BlockSpec((BN, D), lambda g: (g, 0)),
            pl.BlockSpec((D, D), lambda g: (0, 0)),
            pl.BlockSpec((1, D), lambda g: (0, 0)),
            pl.BlockSpec((D, D), lambda g: (0, 0)),
            pl.BlockSpec((D, D), lambda g: (0, 0)),
        ],
        out_specs=[
            pl.BlockSpec((BN, D), lambda g: (g, 0)),
            pl.BlockSpec((BN, D), lambda g: (g, 0)),
            pl.BlockSpec((BN, D), lambda g: (g, 0)),
        ],
        out_shape=[jax.ShapeDtypeStruct((N, D), jnp.float32)] * 3,
    )(x, W_mlp, b_mlp2, W_u, W_v)


def _prep_edge_body(ea_ref, we_ref, o_ref):
    o_ref[...] = jnp.dot(ea_ref[...], we_ref[...],
                         preferred_element_type=jnp.float32) * jnp.float32(1.0 / 3.0)


def _prep_edge(edge_attr, W_edge):
    ed = edge_attr.shape[1]
    return pl.pallas_call(
        _prep_edge_body,
        grid=(NBLK_E,),
        in_specs=[
            pl.BlockSpec((BE, ed), lambda g: (g, 0)),
            pl.BlockSpec((ed, D), lambda g: (0, 0)),
        ],
        out_specs=pl.BlockSpec((BE, D), lambda g: (g, 0)),
        out_shape=jax.ShapeDtypeStruct((E, D), jnp.float32),
    )(edge_attr, W_edge)


@functools.cache
def _gather2_fn():
    return functools.partial(
        pl.kernel,
        out_type=(jax.ShapeDtypeStruct((E, D), jnp.float32),
                  jax.ShapeDtypeStruct((E, D), jnp.float32)),
        mesh=_mesh(),
        compiler_params=pltpu.CompilerParams(needs_layout_passes=False),
        scratch_types=[
            pltpu.VMEM((K,), jnp.int32),
            pltpu.VMEM((K,), jnp.int32),
            pltpu.VMEM((K, D), jnp.float32),
            pltpu.VMEM((K, D), jnp.float32),
            pltpu.SemaphoreType.DMA,
            pltpu.SemaphoreType.DMA,
        ],
    )(_gather2_body)


def _gather2_body(eu_hbm, ev_hbm, src_hbm, dst_hbm, geu_hbm, gev_hbm,
                  src_v, dst_v, bufa, bufb, sema, semb):
    wid = lax.axis_index("s") * NC + lax.axis_index("c")
    nbtot = E // K
    nfull = nbtot // NW
    nb = jnp.where(wid < (nbtot - nfull * NW), nfull + 1, nfull)

    def body(i, _):
        base = (i * NW + wid) * K
        pltpu.sync_copy(src_hbm.at[pl.ds(base, K)], src_v)
        pltpu.sync_copy(dst_hbm.at[pl.ds(base, K)], dst_v)
        ca = pltpu.async_copy(eu_hbm.at[src_v], bufa, sema)
        cb = pltpu.async_copy(ev_hbm.at[dst_v], bufb, semb)
        ca.wait()
        cb.wait()
        pltpu.sync_copy(bufa, geu_hbm.at[pl.ds(base, K)])
        pltpu.sync_copy(bufb, gev_hbm.at[pl.ds(base, K)])
        return ()

    lax.fori_loop(0, nb, body, (), unroll=False)


def _add3_body(a_ref, b_ref, c_ref, o_ref):
    o_ref[...] = a_ref[...] + b_ref[...] + c_ref[...]


def _add3(a, b, c):
    return pl.pallas_call(
        _add3_body,
        grid=(NBLK_E,),
        in_specs=[pl.BlockSpec((BE, D), lambda g: (g, 0))] * 3,
        out_specs=pl.BlockSpec((BE, D), lambda g: (g, 0)),
        out_shape=jax.ShapeDtypeStruct((E, D), jnp.float32),
    )(a, b, c)


@functools.cache
def _lg_pass_fn():
    return functools.partial(
        pl.kernel,
        out_type=jax.ShapeDtypeStruct((E, D), jnp.float32),
        mesh=_mesh(),
        compiler_params=pltpu.CompilerParams(needs_layout_passes=False),
        scratch_types=[
            pltpu.VMEM_SHARED((CH + 8, D), jnp.float32),
            pltpu.VMEM((PIECE,), jnp.int32),
            pltpu.VMEM((PIECE,), jnp.int32),
            pltpu.VMEM((LTP,), jnp.int32),
            pltpu.VMEM((KL,), jnp.int32),
            pltpu.VMEM((KL,), jnp.int32),
            pltpu.VMEM((KL,), jnp.int32),
            pltpu.VMEM((KL,), jnp.int32),
            pltpu.VMEM((KL, D), jnp.float32),
            pltpu.VMEM((KL, D), jnp.float32),
            pltpu.SemaphoreType.DMA,
            pltpu.SemaphoreType.DMA,
            pltpu.SemaphoreType.DMA,
            pltpu.SemaphoreType.DMA,
            pltpu.SemaphoreType.DMA,
        ],
    )(_lg_pass_body)


_SRC_BITS = 18
_PADW = ((CH << _SRC_BITS) & 0xFFFFFFFF) - (1 << 32)


def _lg_pass_body(table_hbm, init_hbm, lgs_hbm, lgd_hbm, out_hbm,
                  shared, dpiece, spiece, pbuf, rel0, src0, rel1, src1,
                  rows0, rows1, semi, semg0, semg1, sems0, sems1):
    cid = lax.axis_index("c")
    sid = lax.axis_index("s")
    trows = 624
    ttail = CH - NS * trows

    def round_body(r, _):
        base = (r * NC + cid) * CH

        pltpu.async_copy(init_hbm.at[pl.ds(base + sid * trows, trows)],
                         shared.at[pl.ds(sid * trows, trows)], semi)

        @pl.when(sid == NS - 1)
        def _init_tail():
            pltpu.async_copy(init_hbm.at[pl.ds(base + NS * trows, ttail)],
                             shared.at[pl.ds(NS * trows, ttail)], semi)

        def piece_body(p, cnt):
            off = sid * LT + p * PIECE
            pltpu.sync_copy(lgd_hbm.at[pl.ds(off, PIECE)], dpiece)
            pltpu.sync_copy(lgs_hbm.at[pl.ds(off, PIECE)], spiece)

            def scan_body(j, cnt2):
                d16 = dpiece[pl.ds(j * 16, 16)]
                s16 = spiece[pl.ds(j * 16, 16)]
                rel = d16 - base
                msk = (rel >= 0) & (rel < CH)
                packed = jnp.where(msk, (rel << _SRC_BITS) | s16,
                                   jnp.int32(_PADW))
                keys = jnp.where(msk, jnp.int32(0), jnp.int32(1))
                _, sv = plsc.sort_key_val(keys, packed)
                pbuf[pl.ds(cnt2, 16)] = sv
                return cnt2 + plsc.all_reduce_population_count(msk)[0]

            return lax.fori_loop(0, PIECE // 16, scan_body, cnt,
                                 unroll=False)

        cnt = lax.fori_loop(0, LT // PIECE, piece_body, jnp.int32(0),
                            unroll=False)

        padv = jnp.full((16,), jnp.int32(_PADW), jnp.int32)
        for t in range(KL // 16):
            pbuf[pl.ds(cnt + t * 16, 16)] = padv
        nb = (cnt + (KL - 1)) // KL

        pltpu.make_async_copy(init_hbm.at[pl.ds(base + sid * trows, trows)],
                              shared.at[pl.ds(sid * trows, trows)],
                              semi).wait()

        @pl.when(sid == NS - 1)
        def _init_tail_wait():
            pltpu.make_async_copy(
                init_hbm.at[pl.ds(base + NS * trows, ttail)],
                shared.at[pl.ds(NS * trows, ttail)], semi).wait()

        plsc.subcore_barrier()

        bufs = ((rel0, src0, rows0, semg0, sems0),
                (rel1, src1, rows1, semg1, sems1))

        def drain_pair(i2, _):
            b0 = i2 * 2
            for j in range(2):
                bi = b0 + j
                rl, sr, rw, sg, ss = bufs[j]

                @pl.when(bi < nb)
                def _issue(bi=bi, rl=rl, sr=sr, rw=rw, sg=sg, ss=ss):
                    @pl.when(bi >= 2)
                    def _reuse_wait():
                        pltpu.make_async_copy(rw, shared.at[rl], ss).wait()

                    for t in range(KL // 16):
                        pk = pbuf[pl.ds(bi * KL + t * 16, 16)]
                        rl[pl.ds(t * 16, 16)] = lax.shift_right_logical(
                            pk, _SRC_BITS)
                        sr[pl.ds(t * 16, 16)] = pk & ((1 << _SRC_BITS) - 1)
                    pltpu.async_copy(table_hbm.at[sr], rw, sg)

            for j in range(2):
                bi = b0 + j
                rl, sr, rw, sg, ss = bufs[j]

                @pl.when(bi < nb)
                def _drain(bi=bi, rl=rl, sr=sr, rw=rw, sg=sg, ss=ss):
                    pltpu.make_async_copy(table_hbm.at[sr], rw, sg).wait()
                    pltpu.async_copy(rw, shared.at[rl], ss, add=True)

            return ()

        lax.fori_loop(0, (nb + 1) // 2, drain_pair, (), unroll=False)

        for j in range(2):
            rl, sr, rw, sg, ss = bufs[j]

            @pl.when(nb > j)
            def _final_drain(rl=rl, rw=rw, ss=ss):
                pltpu.make_async_copy(rw, shared.at[rl], ss).wait()

        plsc.subcore_barrier()

        pltpu.sync_copy(shared.at[pl.ds(sid * trows, trows)],
                        out_hbm.at[pl.ds(base + sid * trows, trows)])

        @pl.when(sid == NS - 1)
        def _out_tail():
            pltpu.sync_copy(shared.at[pl.ds(NS * trows, ttail)],
                            out_hbm.at[pl.ds(base + NS * trows, ttail)])

        return ()

    lax.fori_loop(0, NROUND, round_body, (), unroll=False)


def _xcstats_body(o1, o2, o3, o4, o5, e0r, wr, wl, ba, bat,
                  xct_ref, m_ref, den_ref, m_s, den_s):
    g = pl.program_id(0)

    @pl.when(g == 0)
    def _init():
        m_s[...] = jnp.full((NITER, B), _NEG, jnp.float32)
        den_s[...] = jnp.zeros((NITER, B), jnp.float32)

    outs = (o1[...], o2[...], o3[...], o4[...], o5[...])
    e0b = e0r[...]
    wrv = wr[...]
    wlv = wl[...]
    bav = ba[0, 0]
    ids = bat[0, 0]
    ohT = lax.broadcasted_iota(jnp.int32, (B, BE), 0) == ids[None, :]
    ohTf = ohT.astype(jnp.float32)

    xcs = []
    for n in range(NITER):
        xcn = (jnp.sum(outs[n] * wrv, axis=1)
               + jnp.sum((outs[n + 1] - e0b) * wlv, axis=1) + bav)
        xcs.append(xcn)
        contrib = jnp.max(jnp.where(ohT, xcn[None, :], _NEG), axis=1)
        mold = m_s[n]
        mnew = jnp.maximum(mold, contrib)
        mrow = jnp.dot(mnew, ohTf, preferred_element_type=jnp.float32)
        ex = jnp.exp(xcn - mrow)
        dc = jnp.dot(ohTf, ex, preferred_element_type=jnp.float32)
        den_s[n] = den_s[n] * jnp.exp(mold - mnew) + dc
        m_s[n] = mnew

    xct_ref[...] = jnp.stack(xcs, axis=0)

    @pl.when(g == NBLK_E - 1)
    def _fin():
        m_ref[...] = m_s[...]
        den_ref[...] = den_s[...]


def _xcstats(o1, o2, o3, o4, o5, e0, wr2, wl2, ba2, batch3):
    blk = pl.BlockSpec((BE, D), lambda g: (g, 0))
    return pl.pallas_call(
        _xcstats_body,
        grid=(NBLK_E,),
        in_specs=[blk, blk, blk, blk, blk, blk,
                  pl.BlockSpec((1, D), lambda g: (0, 0)),
                  pl.BlockSpec((1, D), lambda g: (0, 0)),
                  pl.BlockSpec((1, 1), lambda g: (0, 0)),
                  pl.BlockSpec((1, 1, BE), lambda g: (g, 0, 0))],
        out_specs=[pl.BlockSpec((NITER, BE), lambda g: (0, g)),
                   pl.BlockSpec((NITER, B), lambda g: (0, 0)),
                   pl.BlockSpec((NITER, B), lambda g: (0, 0))],
        out_shape=[jax.ShapeDtypeStruct((NITER, E), jnp.float32),
                   jax.ShapeDtypeStruct((NITER, B), jnp.float32),
                   jax.ShapeDtypeStruct((NITER, B), jnp.float32)],
        scratch_shapes=[pltpu.VMEM((NITER, B), jnp.float32),
                        pltpu.VMEM((NITER, B), jnp.float32)],
    )(o1, o2, o3, o4, o5, e0, wr2, wl2, ba2, batch3)


def _gx_body(o1, o2, o3, o4, xct, bat, m, den, wg, bg, a2, ab2,
             sc_ref, gx_s):
    g = pl.program_id(0)

    @pl.when(g == 0)
    def _init():
        gx_s[...] = jnp.zeros((NITER, B, D), jnp.float32)

    outs = (o1[...], o2[...], o3[...], o4[...])
    ids = bat[0, 0]
    ohTf = (lax.broadcasted_iota(jnp.int32, (B, BE), 0)
            == ids[None, :]).astype(jnp.float32)
    for n in range(NITER):
        xcn = xct[n]
        mrow = jnp.dot(m[n], ohTf, preferred_element_type=jnp.float32)
        drow = jnp.dot(den[n], ohTf, preferred_element_type=jnp.float32)
        w = jnp.exp(xcn - mrow) / drow
        gx_s[n] = gx_s[n] + jnp.dot(ohTf, outs[n] * w[:, None],
                                    preferred_element_type=jnp.float32)

    @pl.when(g == NBLK_E - 1)
    def _fin():
        ss = []
        for n in range(NITER):
            gout = jnp.tanh(jnp.dot(gx_s[n], wg[...],
                                    preferred_element_type=jnp.float32) + bg[...])
            ss.append(jnp.sum(gout * a2[n][None, :], axis=1))
        s = jnp.stack(ss, axis=0) + ab2[...]
        smax = jnp.max(s, axis=0)
        e = jnp.exp(s - smax[None, :])
        sc_ref[...] = e / jnp.sum(e, axis=0)[None, :]


def _gx_scores(o1, o2, o3, o4, xct, batch3, m, den, W_gout, bg2, a2, ab2):
    blk = pl.BlockSpec((BE, D), lambda g: (g, 0))
    sml = pl.BlockSpec((NITER, B), lambda g: (0, 0))
    return pl.pallas_call(
        _gx_body,
        grid=(NBLK_E,),
        in_specs=[blk, blk, blk, blk,
                  pl.BlockSpec((NITER, BE), lambda g: (0, g)),
                  pl.BlockSpec((1, 1, BE), lambda g: (g, 0, 0)),
                  sml, sml,
                  pl.BlockSpec((D, D), lambda g: (0, 0)),
                  pl.BlockSpec((1, D), lambda g: (0, 0)),
                  pl.BlockSpec((NITER, D), lambda g: (0, 0)),
                  pl.BlockSpec((NITER, 1), lambda g: (0, 0))],
        out_specs=pl.BlockSpec((NITER, B), lambda g: (0, 0)),
        out_shape=jax.ShapeDtypeStruct((NITER, B), jnp.float32),
        scratch_shapes=[pltpu.VMEM((NITER, B, D), jnp.float32)],
    )(o1, o2, o3, o4, xct, batch3, m, den, W_gout, bg2, a2, ab2)


def _finpool_body(o1, o2, o3, o4, bat, sc, o_ref):
    outs = (o1[...], o2[...], o3[...], o4[...])
    ids = bat[0, 0]
    ohTf = (lax.broadcasted_iota(jnp.int32, (B, BE), 0)
            == ids[None, :]).astype(jnp.float32)
    acc = jnp.zeros((BE, D), jnp.float32)
    for n in range(NITER):
        w = jnp.dot(sc[n], ohTf, preferred_element_type=jnp.float32)
        acc = acc + outs[n] * w[:, None]
    o_ref[...] = acc


def _finpool(o1, o2, o3, o4, batch3, scores):
    blk = pl.BlockSpec((BE, D), lambda g: (g, 0))
    return pl.pallas_call(
        _finpool_body,
        grid=(NBLK_E,),
        in_specs=[blk, blk, blk, blk,
                  pl.BlockSpec((1, 1, BE), lambda g: (g, 0, 0)),
                  pl.BlockSpec((NITER, B), lambda g: (0, 0))],
        out_specs=blk,
        out_shape=jax.ShapeDtypeStruct((E, D), jnp.float32),
    )(o1, o2, o3, o4, batch3, scores)


@functools.cache
def _node_scatter_fn():
    return functools.partial(
        pl.kernel,
        out_type=(jax.ShapeDtypeStruct((N, D), jnp.float32),
                  jax.ShapeDtypeStruct((N, D), jnp.float32)),
        mesh=_mesh(),
        compiler_params=pltpu.CompilerParams(needs_layout_passes=False),
        scratch_types=[
            pltpu.VMEM_SHARED((N, D), jnp.float32),
            pltpu.VMEM((K,), jnp.int32),
            pltpu.VMEM((K, D), jnp.float32),
        ],
    )(_node_scatter_body)


def _node_scatter_body(h_hbm, fin_hbm, dst_hbm, p0_hbm, p1_hbm,
                       shared, idx_stage, rows):
    cid = lax.axis_index("c")
    sid = lax.axis_index("s")
    nrows = 624
    ntail = N - NS * nrows

    pltpu.sync_copy(h_hbm.at[pl.ds(sid * nrows, nrows)],
                    shared.at[pl.ds(sid * nrows, nrows)])

    @pl.when(sid == NS - 1)
    def _init_tail():
        pltpu.sync_copy(h_hbm.at[pl.ds(NS * nrows, ntail)],
                        shared.at[pl.ds(NS * nrows, ntail)])

    plsc.subcore_barrier()

    eh = E // NC
    nbtot = eh // K
    nfull = nbtot // NS
    nb = jnp.where(sid < (nbtot - nfull * NS), nfull + 1, nfull)

    def body(i, _):
        base = cid * eh + (i * NS + sid) * K
        pltpu.sync_copy(dst_hbm.at[pl.ds(base, K)], idx_stage)
        pltpu.sync_copy(fin_hbm.at[pl.ds(base, K)], rows)
        pltpu.sync_copy(rows, shared.at[idx_stage], add=True)
        return ()

    lax.fori_loop(0, nb, body, (), unroll=False)
    plsc.subcore_barrier()

    @pl.when(cid == 0)
    def _w0():
        pltpu.sync_copy(shared.at[pl.ds(sid * nrows, nrows)],
                        p0_hbm.at[pl.ds(sid * nrows, nrows)])

        @pl.when(sid == NS - 1)
        def _w0t():
            pltpu.sync_copy(shared.at[pl.ds(NS * nrows, ntail)],
                            p0_hbm.at[pl.ds(NS * nrows, ntail)])

    @pl.when(cid == 1)
    def _w1():
        pltpu.sync_copy(shared.at[pl.ds(sid * nrows, nrows)],
                        p1_hbm.at[pl.ds(sid * nrows, nrows)])

        @pl.when(sid == NS - 1)
        def _w1t():
            pltpu.sync_copy(shared.at[pl.ds(NS * nrows, ntail)],
                            p1_hbm.at[pl.ds(NS * nrows, ntail)])


def _final_body(p0, p1, hr, wl_ref, bl_ref, o_ref):
    xo = p0[...] + p1[...] - hr[...]
    o_ref[...] = jnp.dot(xo, wl_ref[...],
                         preferred_element_type=jnp.float32) + bl_ref[...]


def _final(p0, p1, h, W_lb, bl2):
    return pl.pallas_call(
        _final_body,
        grid=(NBLK_N,),
        in_specs=[pl.BlockSpec((BN, D), lambda g: (g, 0))] * 3 +
                 [pl.BlockSpec((D, D), lambda g: (0, 0)),
                  pl.BlockSpec((1, D), lambda g: (0, 0))],
        out_specs=pl.BlockSpec((BN, D), lambda g: (g, 0)),
        out_shape=jax.ShapeDtypeStruct((N, D), jnp.float32),
    )(p0, p1, h, W_lb, bl2)


def kernel(x, edge_attr, edge_index, line_graph_edge_index, edge_index_batch,
           W_mlp, b_mlp, W_u, W_v, W_edge, W_att_root, W_att_rel, b_att, a,
           W_gout, b_gout, a_bias, W_lb, b_lb):
    src = edge_index[0].astype(jnp.int32)
    dst = edge_index[1].astype(jnp.int32)
    lgs = line_graph_edge_index[0].astype(jnp.int32)
    lgd = line_graph_edge_index[1].astype(jnp.int32)
    batch3 = edge_index_batch.astype(jnp.int32).reshape(NBLK_E, 1, BE)

    h, eu3, ev3 = _prep_node(x, W_mlp, b_mlp.reshape(1, D), W_u, W_v)
    euv3 = _prep_edge(edge_attr, W_edge)
    geu, gev = _gather2_fn()(eu3, ev3, src, dst)
    e0 = _add3(geu, gev, euv3)

    outs = [e0]
    for _ in range(NITER + 1):
        outs.append(_lg_pass_fn()(outs[-1], e0, lgs, lgd))

    wr2 = W_att_root.reshape(1, D)
    wl2 = W_att_rel.reshape(1, D)
    ba2 = b_att.reshape(1, 1)
    xct, m, den = _xcstats(outs[1], outs[2], outs[3], outs[4], outs[5],
                           e0, wr2, wl2, ba2, batch3)

    a2 = jnp.transpose(a[0])
    ab2 = a_bias.reshape(NITER, 1)
    scores = _gx_scores(outs[1], outs[2], outs[3], outs[4], xct, batch3,
                        m, den, W_gout, b_gout.reshape(1, D), a2, ab2)

    out_fin = _finpool(outs[1], outs[2], outs[3], outs[4], batch3, scores)

    p0, p1 = _node_scatter_fn()(h, out_fin, dst)
    return _final(p0, p1, h, W_lb, b_lb.reshape(1, D))
